# baseline (speedup 1.0000x reference)
import functools

import jax
import jax.numpy as jnp
from jax import lax
from jax.experimental import pallas as pl
from jax.experimental.pallas import tpu as pltpu
from jax.experimental.pallas import tpu_sc as plsc

_POS_TH = 2.0
_B = 8
_N = 512 * 512
_M = _N // 4
_CHUNK = 8192
_NCH = _M // _CHUNK
_CV = _CHUNK // 16
_NV = _M // 16
_NB = 256
_SHIFTS = (23, 15, 7, 0)
_LBITS = (8, 8, 8, 7)


def _sc_body(x_hbm, y_hbm, out_hbm, nb, xb, yb, hist, shist, comb,
             thist, tshist, tcf, statv, outv, sh_all):
    cid = lax.axis_index("c")
    sid = lax.axis_index("s")
    sample = cid * 4 + sid // 4
    quarter = sid % 4
    q0 = (sid // 4) * 4
    base = pl.multiple_of(sample * _N + quarter * _M, 8)

    iot = lax.iota(jnp.int32, 16)
    zero_i = jnp.zeros((16,), jnp.int32)
    zero_f = jnp.zeros((16,), jnp.float32)
    ones_i = jnp.ones((16,), jnp.int32)

    def zero_hists(j, _):
        for u in range(4):
            hist[pl.ds((j * 4 + u) * 16, 16)] = zero_i
            shist[pl.ds((j * 4 + u) * 16, 16)] = zero_f
        return 0

    lax.fori_loop(0, _NB * 16 // 64, zero_hists, 0)

    _U = 4
    sh0 = jnp.full((16,), _SHIFTS[0], jnp.int32)
    accs = (zero_i, zero_f, zero_f) * _U
    for c in range(_NCH):
        off = pl.multiple_of(base + c * _CHUNK, 8)
        pltpu.sync_copy(x_hbm.at[pl.ds(off, _CHUNK)], xb)
        pltpu.sync_copy(y_hbm.at[pl.ds(off, _CHUNK)], yb)

        def p1_body(i, acc, c=c):
            out = []
            for u in range(_U):
                ap, aps, at = acc[3 * u : 3 * u + 3]
                s = (i * _U + u) * 16
                xv = xb[pl.ds(s, 16)]
                yv = yb[pl.ds(s, 16)]
                d = xv - yv
                lv = d * d
                posm = yv > _POS_TH
                ap = ap + jnp.where(posm, ones_i, zero_i)
                aps = aps + jnp.where(posm, lv, zero_f)
                at = at + lv
                nbv = jnp.where(
                    posm, zero_i, lax.bitcast_convert_type(lv, jnp.int32)
                )
                nb[pl.ds(c * _CHUNK + s, 16)] = nbv
                idx = iot * _NB + lax.shift_right_logical(nbv, sh0)
                plsc.addupdate_scatter(hist, [idx], ones_i)
                plsc.addupdate_scatter(
                    shist, [idx], jnp.where(posm, zero_f, lv)
                )
                out.extend((ap, aps, at))
            return tuple(out)

        accs = lax.fori_loop(0, _CV // _U, p1_body, accs)
    ap = accs[0] + accs[3] + accs[6] + accs[9]
    aps = accs[1] + accs[4] + accs[7] + accs[10]
    at = accs[2] + accs[5] + accs[8] + accs[11]

    def lane_reduce(nbins):
        def body(j, _):
            acc_c = zero_i
            acc_s = zero_f
            for l in range(16):
                acc_c = acc_c + hist[pl.ds(l * _NB + j * 16, 16)]
                acc_s = acc_s + shist[pl.ds(l * _NB + j * 16, 16)]
            thist[pl.ds(j * 16, 16)] = acc_c
            tshist[pl.ds(j * 16, 16)] = acc_s
            return 0

        lax.fori_loop(0, nbins // 16, body, 0)

    def publish_combine(nbins):
        def cvt(j, _):
            tcf[pl.ds(j * 16, 16)] = thist[pl.ds(j * 16, 16)].astype(
                jnp.float32
            )
            return 0

        lax.fori_loop(0, nbins // 16, cvt, 0)
        pltpu.sync_copy(tcf, sh_all.at[pl.ds(768 * sid, _NB)])
        pltpu.sync_copy(tshist, sh_all.at[pl.ds(768 * sid + _NB, _NB)])
        plsc.subcore_barrier()
        pltpu.sync_copy(sh_all.at[pl.ds(768 * q0, 3072)], comb)
        plsc.subcore_barrier()

        def body(j, _):
            acc_c = zero_f
            acc_s = zero_f
            for r in range(4):
                acc_c = acc_c + comb[pl.ds(768 * r + j * 16, 16)]
                acc_s = acc_s + comb[pl.ds(768 * r + _NB + j * 16, 16)]
            thist[pl.ds(j * 16, 16)] = acc_c.astype(jnp.int32)
            tshist[pl.ds(j * 16, 16)] = acc_s
            return 0

        lax.fori_loop(0, nbins // 16, body, 0)

    def cnt_ge(e, nbins):
        def body(j, acc):
            lbl = j * 16 + iot
            return acc + jnp.where(lbl >= e, thist[pl.ds(j * 16, 16)], zero_i)

        return jnp.sum(lax.fori_loop(0, nbins // 16, body, zero_i))

    def sum_ge_vec(e, nbins):
        def body(j, acc):
            lbl = j * 16 + iot
            return acc + jnp.where(lbl >= e, tshist[pl.ds(j * 16, 16)], zero_f)

        return lax.fori_loop(0, nbins // 16, body, zero_f)

    def search(k_rem, nbits):
        nbins = 1 << nbits

        def body(_, c):
            lo, hi = c
            mid = lo + (hi - lo) // 2
            ok = cnt_ge(mid, nbins) >= k_rem
            return jnp.where(ok, mid, lo), jnp.where(ok, hi, mid)

        lo, _ = lax.fori_loop(
            0, nbits, body, (jnp.int32(0), jnp.int32(nbins))
        )
        return lo, cnt_ge(lo + 1, nbins), sum_ge_vec(lo + 1, nbins)

    lane_reduce(1 << _LBITS[0])
    statv[pl.ds(0, 16)] = ap.astype(jnp.float32)
    statv[pl.ds(16, 16)] = aps
    statv[pl.ds(32, 16)] = at
    pltpu.sync_copy(statv, sh_all.at[pl.ds(768 * sid + 2 * _NB, _NB)])
    publish_combine(1 << _LBITS[0])

    pv = zero_f
    psv = zero_f
    tv = zero_f
    for r in range(4):
        pv = pv + comb[pl.ds(768 * r + 2 * _NB, 16)]
        psv = psv + comb[pl.ds(768 * r + 2 * _NB + 16, 16)]
        tv = tv + comb[pl.ds(768 * r + 2 * _NB + 32, 16)]
    p_i = jnp.sum(pv.astype(jnp.int32))
    k_i = 3 * p_i

    b1, ac1, asv1 = search(k_i, _LBITS[0])
    prefix = b1
    k_rem = k_i - ac1
    above_cnt = ac1
    asum_v = asv1

    for lvl in range(1, 4):
        nbits = _LBITS[lvl]
        nbins = 1 << nbits
        lax.fori_loop(0, _NB * 16 // 64, zero_hists, 0)
        pv_prefix = jnp.broadcast_to(prefix, (16,))
        shp = jnp.full((16,), _SHIFTS[lvl - 1], jnp.int32)
        shc = jnp.full((16,), _SHIFTS[lvl], jnp.int32)
        bmask = jnp.full((16,), nbins - 1, jnp.int32)

        def hist_body(i, _):
            for u in range(_U):
                v = nb[pl.ds((i * _U + u) * 16, 16)]
                m = lax.shift_right_logical(v, shp) == pv_prefix
                bn = jnp.bitwise_and(lax.shift_right_logical(v, shc), bmask)
                idx = iot * _NB + bn
                plsc.addupdate_scatter(hist, [idx], ones_i, mask=m)
                plsc.addupdate_scatter(
                    shist,
                    [idx],
                    lax.bitcast_convert_type(v, jnp.float32),
                    mask=m,
                )
            return 0

        lax.fori_loop(0, _NV // _U, hist_body, 0)
        lane_reduce(nbins)
        publish_combine(nbins)
        b, ac, asv = search(k_rem, nbits)
        prefix = prefix * nbins + b
        k_rem = k_rem - ac
        above_cnt = above_cnt + ac
        asum_v = asum_v + asv

    t_vec = lax.bitcast_convert_type(jnp.broadcast_to(prefix, (16,)), jnp.float32)
    kf_v = jnp.broadcast_to(k_i, (16,)).astype(jnp.float32)
    pf_v = jnp.broadcast_to(p_i, (16,)).astype(jnp.float32)
    cgt_v = jnp.broadcast_to(above_cnt, (16,)).astype(jnp.float32)
    sum_gt_v = jnp.broadcast_to(jnp.sum(asum_v), (16,))
    pos_sum_v = jnp.broadcast_to(jnp.sum(psv), (16,))
    total_v = jnp.broadcast_to(jnp.sum(tv), (16,))

    topk_v = sum_gt_v + (kf_v - cgt_v) * t_vec
    fallback_v = total_v * (1.0 / _N)
    mined_v = pos_sum_v / jnp.maximum(pf_v, 1.0) + topk_v / jnp.maximum(
        kf_v, 1.0
    )
    cond = (k_i + p_i >= _N) | (k_i <= 10)
    outv[...] = jnp.where(cond, fallback_v, mined_v)

    @pl.when(quarter == 0)
    def _():
        pltpu.sync_copy(outv, out_hbm.at[sample])


_sc_kernel = functools.partial(
    pl.kernel,
    out_type=jax.ShapeDtypeStruct((_B, 16), jnp.float32),
    mesh=plsc.VectorSubcoreMesh(core_axis_name="c", subcore_axis_name="s"),
    compiler_params=pltpu.CompilerParams(needs_layout_passes=False),
    scratch_types=[
        pltpu.VMEM((_M,), jnp.int32),
        pltpu.VMEM((_CHUNK,), jnp.float32),
        pltpu.VMEM((_CHUNK,), jnp.float32),
        pltpu.VMEM((_NB * 16,), jnp.int32),
        pltpu.VMEM((_NB * 16,), jnp.float32),
        pltpu.VMEM((3072,), jnp.float32),
        pltpu.VMEM((_NB,), jnp.int32),
        pltpu.VMEM((_NB,), jnp.float32),
        pltpu.VMEM((_NB,), jnp.float32),
        pltpu.VMEM((_NB,), jnp.float32),
        pltpu.VMEM((16,), jnp.float32),
        pltpu.VMEM_SHARED((12288,), jnp.float32),
    ],
)(_sc_body)


def kernel(x, y):
    out = _sc_kernel(x.reshape(-1), y.reshape(-1))
    return jnp.mean(out[:, 0])


# speedup vs baseline: 2.0577x; 2.0577x over previous
import functools

import jax
import jax.numpy as jnp
from jax import lax
from jax.experimental import pallas as pl
from jax.experimental.pallas import tpu as pltpu
from jax.experimental.pallas import tpu_sc as plsc

_POS_TH = 2.0
_B = 8
_N = 512 * 512
_M = _N // 4
_CHUNK = 8192
_NCH = _M // _CHUNK
_CV = _CHUNK // 16
_NV = _M // 16
_NB = 256
_SHIFTS = (23, 15, 7, 0)
_LBITS = (8, 8, 8, 7)


def _sc_body(x_hbm, y_hbm, out_hbm, nb, xb, yb, hist, shist, comb,
             thist, tshist, tcf, statv, outv, sh_all):
    cid = lax.axis_index("c")
    sid = lax.axis_index("s")
    sample = cid * 4 + sid // 4
    quarter = sid % 4
    q0 = (sid // 4) * 4
    base = pl.multiple_of(sample * _N + quarter * _M, 8)

    iot = lax.iota(jnp.int32, 16)
    zero_i = jnp.zeros((16,), jnp.int32)
    zero_f = jnp.zeros((16,), jnp.float32)
    ones_i = jnp.ones((16,), jnp.int32)

    def zero_body(j):
        for u in range(4):
            hist[pl.ds(j + u * 16, 16)] = zero_i
            shist[pl.ds(j + u * 16, 16)] = zero_f

    plsc.parallel_loop(0, _NB * 16, step=64)(zero_body)

    _U = 4
    sh0 = jnp.full((16,), _SHIFTS[0], jnp.int32)
    apc = (zero_i,) * _U
    for c in range(_NCH):
        off = pl.multiple_of(base + c * _CHUNK, 8)
        pltpu.sync_copy(x_hbm.at[pl.ds(off, _CHUNK)], xb)
        pltpu.sync_copy(y_hbm.at[pl.ds(off, _CHUNK)], yb)

        def p1_body(i, acc, c=c):
            out = []
            for u in range(_U):
                s = i + u * 16
                xv = xb[pl.ds(s, 16)]
                yv = yb[pl.ds(s, 16)]
                d = xv - yv
                lv = d * d
                posm = yv > _POS_TH
                nbv = jnp.where(
                    posm, zero_i, lax.bitcast_convert_type(lv, jnp.int32)
                )
                nb[pl.ds(c * _CHUNK + s, 16)] = nbv
                idx = iot * _NB + lax.shift_right_logical(nbv, sh0)
                plsc.addupdate_scatter(hist, [idx], ones_i)
                plsc.addupdate_scatter(shist, [idx], lv)
                out.append(acc[u] + jnp.where(posm, ones_i, zero_i))
            return tuple(out)

        apc = plsc.parallel_loop(0, _CHUNK, step=16 * _U, carry=apc)(p1_body)
    ap = apc[0] + apc[1] + apc[2] + apc[3]

    def lane_reduce(nbins):
        def body(j, _):
            acc_c = zero_i
            acc_s = zero_f
            for l in range(16):
                acc_c = acc_c + hist[pl.ds(l * _NB + j * 16, 16)]
                acc_s = acc_s + shist[pl.ds(l * _NB + j * 16, 16)]
            thist[pl.ds(j * 16, 16)] = acc_c
            tshist[pl.ds(j * 16, 16)] = acc_s
            return 0

        lax.fori_loop(0, nbins // 16, body, 0)

    def publish_combine(nbins):
        def cvt(j, _):
            tcf[pl.ds(j * 16, 16)] = thist[pl.ds(j * 16, 16)].astype(
                jnp.float32
            )
            return 0

        lax.fori_loop(0, nbins // 16, cvt, 0)
        pltpu.sync_copy(tcf, sh_all.at[pl.ds(768 * sid, _NB)])
        pltpu.sync_copy(tshist, sh_all.at[pl.ds(768 * sid + _NB, _NB)])
        plsc.subcore_barrier()
        pltpu.sync_copy(sh_all.at[pl.ds(768 * q0, 3072)], comb)
        plsc.subcore_barrier()

        def body(j, _):
            acc_c = zero_f
            acc_s = zero_f
            for r in range(4):
                acc_c = acc_c + comb[pl.ds(768 * r + j * 16, 16)]
                acc_s = acc_s + comb[pl.ds(768 * r + _NB + j * 16, 16)]
            thist[pl.ds(j * 16, 16)] = acc_c.astype(jnp.int32)
            tshist[pl.ds(j * 16, 16)] = acc_s
            return 0

        lax.fori_loop(0, nbins // 16, body, 0)

    def cnt_ge(e, nbins):
        def body(j, acc):
            lbl = j * 16 + iot
            return acc + jnp.where(lbl >= e, thist[pl.ds(j * 16, 16)], zero_i)

        return jnp.sum(lax.fori_loop(0, nbins // 16, body, zero_i))

    def sum_ge_vec(e, nbins):
        def body(j, acc):
            lbl = j * 16 + iot
            return acc + jnp.where(lbl >= e, tshist[pl.ds(j * 16, 16)], zero_f)

        return lax.fori_loop(0, nbins // 16, body, zero_f)

    def search(k_rem, nbits):
        nbins = 1 << nbits

        def body(_, c):
            lo, hi = c
            mid = lo + (hi - lo) // 2
            ok = cnt_ge(mid, nbins) >= k_rem
            return jnp.where(ok, mid, lo), jnp.where(ok, hi, mid)

        lo, _ = lax.fori_loop(
            0, nbits, body, (jnp.int32(0), jnp.int32(nbins))
        )
        return lo, cnt_ge(lo + 1, nbins), sum_ge_vec(lo + 1, nbins)

    lane_reduce(1 << _LBITS[0])
    statv[pl.ds(0, 16)] = ap.astype(jnp.float32)
    pltpu.sync_copy(statv, sh_all.at[pl.ds(768 * sid + 2 * _NB, _NB)])
    publish_combine(1 << _LBITS[0])

    pv = zero_f
    for r in range(4):
        pv = pv + comb[pl.ds(768 * r + 2 * _NB, 16)]
    p_i = jnp.sum(pv.astype(jnp.int32))
    k_i = 3 * p_i

    pos_sum = jnp.sum(jnp.where(iot == 0, tshist[pl.ds(0, 16)], zero_f))
    tacc = zero_f
    for j in range(16):
        tacc = tacc + tshist[pl.ds(j * 16, 16)]
    total = jnp.sum(tacc)

    b1, ac1, asv1 = search(k_i, _LBITS[0])
    prefix = b1
    k_rem = k_i - ac1
    above_cnt = ac1
    asum_v = asv1

    for lvl in range(1, 4):
        nbits = _LBITS[lvl]
        nbins = 1 << nbits
        plsc.parallel_loop(0, _NB * 16, step=64)(zero_body)
        pv_prefix = jnp.broadcast_to(prefix, (16,))
        shp = jnp.full((16,), _SHIFTS[lvl - 1], jnp.int32)
        shc = jnp.full((16,), _SHIFTS[lvl], jnp.int32)
        bmask = jnp.full((16,), nbins - 1, jnp.int32)

        def hist_body(i):
            for u in range(_U):
                v = nb[pl.ds(i + u * 16, 16)]
                m = lax.shift_right_logical(v, shp) == pv_prefix
                bn = jnp.bitwise_and(lax.shift_right_logical(v, shc), bmask)
                idx = iot * _NB + bn
                plsc.addupdate_scatter(hist, [idx], ones_i, mask=m)
                plsc.addupdate_scatter(
                    shist,
                    [idx],
                    lax.bitcast_convert_type(v, jnp.float32),
                    mask=m,
                )

        plsc.parallel_loop(0, _M, step=16 * _U)(hist_body)
        lane_reduce(nbins)
        publish_combine(nbins)
        b, ac, asv = search(k_rem, nbits)
        prefix = prefix * nbins + b
        k_rem = k_rem - ac
        above_cnt = above_cnt + ac
        asum_v = asum_v + asv

    t_vec = lax.bitcast_convert_type(jnp.broadcast_to(prefix, (16,)), jnp.float32)
    kf_v = jnp.broadcast_to(k_i, (16,)).astype(jnp.float32)
    pf_v = jnp.broadcast_to(p_i, (16,)).astype(jnp.float32)
    cgt_v = jnp.broadcast_to(above_cnt, (16,)).astype(jnp.float32)
    sum_gt_v = jnp.broadcast_to(jnp.sum(asum_v), (16,))
    pos_sum_v = jnp.broadcast_to(pos_sum, (16,))
    total_v = jnp.broadcast_to(total, (16,))

    topk_v = sum_gt_v + (kf_v - cgt_v) * t_vec
    fallback_v = total_v * (1.0 / _N)
    mined_v = pos_sum_v / jnp.maximum(pf_v, 1.0) + topk_v / jnp.maximum(
        kf_v, 1.0
    )
    cond = (k_i + p_i >= _N) | (k_i <= 10)
    outv[...] = jnp.where(cond, fallback_v, mined_v)

    @pl.when(quarter == 0)
    def _():
        pltpu.sync_copy(outv, out_hbm.at[sample])


_sc_kernel = functools.partial(
    pl.kernel,
    out_type=jax.ShapeDtypeStruct((_B, 16), jnp.float32),
    mesh=plsc.VectorSubcoreMesh(core_axis_name="c", subcore_axis_name="s"),
    compiler_params=pltpu.CompilerParams(needs_layout_passes=False),
    scratch_types=[
        pltpu.VMEM((_M,), jnp.int32),
        pltpu.VMEM((_CHUNK,), jnp.float32),
        pltpu.VMEM((_CHUNK,), jnp.float32),
        pltpu.VMEM((_NB * 16,), jnp.int32),
        pltpu.VMEM((_NB * 16,), jnp.float32),
        pltpu.VMEM((3072,), jnp.float32),
        pltpu.VMEM((_NB,), jnp.int32),
        pltpu.VMEM((_NB,), jnp.float32),
        pltpu.VMEM((_NB,), jnp.float32),
        pltpu.VMEM((_NB,), jnp.float32),
        pltpu.VMEM((16,), jnp.float32),
        pltpu.VMEM_SHARED((12288,), jnp.float32),
    ],
)(_sc_body)


def kernel(x, y):
    out = _sc_kernel(x.reshape(-1), y.reshape(-1))
    return jnp.mean(out[:, 0])
